# Initial kernel scaffold; baseline (speedup 1.0000x reference)
import jax
import jax.numpy as jnp
from jax.experimental import pallas as pl


def kernel(x, edge_index, W_self, W_neigh, b_enc, W_dec, b_dec):
    raise NotImplementedError("write your pallas kernel here")


# speedup vs baseline: 7.9372x; 7.9372x over previous
import functools

import jax
import jax.numpy as jnp
from jax import lax
from jax.experimental import pallas as pl
from jax.experimental.pallas import tpu as pltpu
from jax.experimental.pallas import tpu_sc as plsc

N_NODES = 10000
N_EDGES = 320000
D = 128

NC = 2
NS = 16
NW = NC * NS
E_PER_W = N_EDGES // NW
CHUNK = 80
NCHUNK = E_PER_W // CHUNK
ROWS_PER_S = N_NODES // NS
DEG_W = 16


def _sc_body(x_hbm, src_hbm, dst_hbm, acc_out, deg_out,
             acc_sh, deg_sh, src_all, dst_all, rows_v, ones_v,
             zbuf, sem):
    cid = lax.axis_index("c")
    sid = lax.axis_index("s")

    zero16 = jnp.zeros((16,), jnp.float32)
    one16 = jnp.ones((16,), jnp.float32)

    def _fill(r, _):
        for cc in range(D // 16):
            rows_v[r, pl.ds(cc * 16, 16)] = zero16
        ones_v[r, :] = one16
        zbuf[r, :] = zero16
        return _
    lax.fori_loop(0, CHUNK, _fill, 0)

    row0 = sid * ROWS_PER_S
    for k in range(ROWS_PER_S // CHUNK):
        pltpu.sync_copy(rows_v, acc_sh.at[pl.ds(row0 + k * CHUNK, CHUNK)])
        pltpu.sync_copy(zbuf, deg_sh.at[pl.ds(row0 + k * CHUNK, CHUNK)])
    rem = ROWS_PER_S % CHUNK
    base = row0 + (ROWS_PER_S // CHUNK) * CHUNK
    pltpu.sync_copy(rows_v.at[pl.ds(0, rem)], acc_sh.at[pl.ds(base, rem)])
    pltpu.sync_copy(zbuf.at[pl.ds(0, rem)], deg_sh.at[pl.ds(base, rem)])

    chunk0 = (cid * NS + sid) * NCHUNK
    pltpu.sync_copy(src_hbm.at[pl.ds(chunk0, NCHUNK)], src_all)
    pltpu.sync_copy(dst_hbm.at[pl.ds(chunk0, NCHUNK)], dst_all)

    plsc.subcore_barrier()

    def _edge_chunk(j, _):
        pltpu.async_copy(x_hbm.at[src_all.at[j]], rows_v, sem).wait()
        pltpu.sync_copy(rows_v, acc_sh.at[dst_all.at[j]], add=True)
        pltpu.sync_copy(ones_v, deg_sh.at[dst_all.at[j]], add=True)
        return _
    lax.fori_loop(0, NCHUNK, _edge_chunk, 0)

    plsc.subcore_barrier()

    pltpu.sync_copy(acc_sh.at[pl.ds(row0, ROWS_PER_S)],
                    acc_out.at[cid, pl.ds(row0, ROWS_PER_S)])
    pltpu.sync_copy(deg_sh.at[pl.ds(row0, ROWS_PER_S)],
                    deg_out.at[cid, pl.ds(row0, ROWS_PER_S)])


@functools.partial(
    pl.kernel,
    out_type=[
        jax.ShapeDtypeStruct((NC, N_NODES, D), jnp.float32),
        jax.ShapeDtypeStruct((NC, N_NODES, DEG_W), jnp.float32),
    ],
    mesh=plsc.VectorSubcoreMesh(core_axis_name="c", subcore_axis_name="s"),
    scratch_types=[
        pltpu.VMEM_SHARED((N_NODES, D), jnp.float32),
        pltpu.VMEM_SHARED((N_NODES, DEG_W), jnp.float32),
        pltpu.VMEM((NCHUNK, CHUNK), jnp.int32),
        pltpu.VMEM((NCHUNK, CHUNK), jnp.int32),
        pltpu.VMEM((CHUNK, D), jnp.float32),
        pltpu.VMEM((CHUNK, DEG_W), jnp.float32),
        pltpu.VMEM((CHUNK, DEG_W), jnp.float32),
        pltpu.SemaphoreType.DMA,
    ],
    compiler_params=pltpu.CompilerParams(use_tc_tiling_on_sc=False),
)
def _sc_aggregate(x_hbm, src_hbm, dst_hbm, acc_out, deg_out, *scratch):
    _sc_body(x_hbm, src_hbm, dst_hbm, acc_out, deg_out, *scratch)


def _tc_decode_body(x_ref, p0_ref, p1_ref, d0_ref, d1_ref, ws_ref, wn_ref,
                    be_ref, wd_ref, bd_ref, rec_ref, h_ref):
    summed = p0_ref[...] + p1_ref[...]
    deg = jnp.maximum(d0_ref[...] + d1_ref[...], 1.0)
    h_neigh = summed / deg
    h_enc = (jnp.dot(x_ref[...], ws_ref[...], preferred_element_type=jnp.float32)
             + jnp.dot(h_neigh, wn_ref[...], preferred_element_type=jnp.float32)
             + be_ref[...])
    h = jnp.maximum(h_enc, 0.0)
    h_ref[...] = h
    rec_ref[...] = (jnp.dot(h, wd_ref[...], preferred_element_type=jnp.float32)
                    + bd_ref[...])


def _tc_decode(x, p0, p1, d0, d1, W_self, W_neigh, b_enc, W_dec, b_dec):
    R = 1000
    grid = (N_NODES // R,)
    row_spec = pl.BlockSpec((R, D), lambda i: (i, 0))
    deg_spec = pl.BlockSpec((R, 1), lambda i: (i, 0))
    w_spec = pl.BlockSpec((D, D), lambda i: (0, 0))
    b_spec = pl.BlockSpec((1, D), lambda i: (0, 0))
    return pl.pallas_call(
        _tc_decode_body,
        grid=grid,
        in_specs=[row_spec, row_spec, row_spec, deg_spec, deg_spec,
                  w_spec, w_spec, b_spec, w_spec, b_spec],
        out_specs=[row_spec, row_spec],
        out_shape=[
            jax.ShapeDtypeStruct((N_NODES, D), jnp.float32),
            jax.ShapeDtypeStruct((N_NODES, D), jnp.float32),
        ],
    )(x, p0, p1, d0, d1, W_self, W_neigh, b_enc, W_dec, b_dec)


@jax.jit
def kernel(x, edge_index, W_self, W_neigh, b_enc, W_dec, b_dec):
    src = edge_index[0].astype(jnp.int32).reshape(NW * NCHUNK, CHUNK)
    dst = edge_index[1].astype(jnp.int32).reshape(NW * NCHUNK, CHUNK)
    acc, deg = _sc_aggregate(x, src, dst)
    rec, h = _tc_decode(
        x, acc[0], acc[1], deg[0, :, 0:1], deg[1, :, 0:1],
        W_self, W_neigh, b_enc.reshape(1, D), W_dec, b_dec.reshape(1, D))
    return rec, h
